# baseline (speedup 1.0000x reference)
import functools

import jax
import jax.numpy as jnp
from jax import lax
from jax.experimental import pallas as pl
from jax.experimental.pallas import tpu as pltpu
from jax.experimental.pallas import tpu_sc as plsc

HIDDEN = 2048
NUM_EXPERTS = 16
TOP_K = 2
EPS = 1e-6
TOKENS = 8192

BLOCK_T = 1024

_SC_INFO = plsc.get_sparse_core_info()
_NC, _NS, _NL = _SC_INFO.num_cores, _SC_INFO.num_subcores, _SC_INFO.num_lanes
_NW = _NC * _NS
_TOK_W = TOKENS // _NW
_GROUPS = _TOK_W // _NL


def _dense_body(xb, scale_row, w):
    ms = jnp.mean(xb * xb, axis=-1, keepdims=True)
    y = xb * lax.rsqrt(ms + EPS)
    y = y * scale_row
    y = y * (HIDDEN ** -0.5)
    logits = lax.dot_general(
        y, w,
        dimension_numbers=(((1,), (1,)), ((), ())),
        preferred_element_type=jnp.float32,
    )
    m = jnp.max(logits, axis=-1, keepdims=True)
    e = jnp.exp(logits - m)
    return e / jnp.sum(e, axis=-1, keepdims=True)


def _dense_block(x_ref, scale_ref, w_ref, p_ref):
    p_ref[...] = _dense_body(x_ref[...], scale_ref[...], w_ref[...])


_TOKW = TOKENS // _NW
_NGROUPS = _TOKW // _NL


def _sc_router(probs_hbm, pes_hbm, idx_hbm, wgt_hbm,
               pv, pesv, isrc, wsrc, iout, wout, sem_p, sem_s):
    wid = lax.axis_index("s") * _NC + lax.axis_index("c")
    base = wid * _TOKW
    cp_p = pltpu.async_copy(probs_hbm.at[pl.ds(base, _TOKW), :], pv, sem_p)
    cp_s = pltpu.async_copy(pes_hbm, pesv, sem_s)
    cp_p.wait()
    cp_s.wait()
    lanes = lax.iota(jnp.int32, _NL)

    @plsc.parallel_loop(0, _NGROUPS, unroll=2)
    def group_body(g):
        rows = g * _NL + lanes
        m1 = jnp.full((_NL,), -jnp.inf, jnp.float32)
        m2 = jnp.full((_NL,), -jnp.inf, jnp.float32)
        i1 = jnp.zeros((_NL,), jnp.int32)
        i2 = jnp.zeros((_NL,), jnp.int32)
        for e in range(NUM_EXPERTS):
            v = plsc.load_gather(pv, [rows, jnp.full((_NL,), e, jnp.int32)])
            is1 = v > m1
            is2 = jnp.logical_not(is1) & (v > m2)
            m2 = jnp.where(is1, m1, jnp.where(is2, v, m2))
            i2 = jnp.where(is1, i1, jnp.where(is2, e, i2))
            m1 = jnp.where(is1, v, m1)
            i1 = jnp.where(is1, e, i1)
        g1 = plsc.load_gather(pesv, [i1])
        g2 = plsc.load_gather(pesv, [i2])
        s = m1 + m2
        sl = pl.ds(g * _NL, _NL)
        isrc[0, sl] = i1
        isrc[1, sl] = i2
        wsrc[0, sl] = (m1 / s) * g1
        wsrc[1, sl] = (m2 / s) * g2

    fld = jnp.bitwise_and(lanes, 1)

    @plsc.parallel_loop(0, (_TOKW * 2) // _NL, unroll=2)
    def interleave_body(j):
        tok = j * (_NL // 2) + lax.shift_right_logical(lanes, 1)
        sl = pl.ds(j * _NL, _NL)
        iout[sl] = plsc.load_gather(isrc, [fld, tok])
        wout[sl] = plsc.load_gather(wsrc, [fld, tok])

    out_sl = pl.ds(base * 2, _TOKW * 2)
    pltpu.sync_copy(iout, idx_hbm.at[out_sl])
    pltpu.sync_copy(wout, wgt_hbm.at[out_sl])


_sc_router_full = functools.partial(
    pl.kernel,
    out_type=[
        jax.ShapeDtypeStruct((TOKENS * 2,), jnp.int32),
        jax.ShapeDtypeStruct((TOKENS * 2,), jnp.float32),
    ],
    mesh=plsc.VectorSubcoreMesh(core_axis_name="c", subcore_axis_name="s"),
    compiler_params=pltpu.CompilerParams(needs_layout_passes=False),
    scratch_types=[
        pltpu.VMEM((_TOKW, NUM_EXPERTS), jnp.float32),
        pltpu.VMEM((NUM_EXPERTS,), jnp.float32),
        pltpu.VMEM((2, _TOKW), jnp.int32),
        pltpu.VMEM((2, _TOKW), jnp.float32),
        pltpu.VMEM((_TOKW * 2,), jnp.int32),
        pltpu.VMEM((_TOKW * 2,), jnp.float32),
        pltpu.SemaphoreType.DMA,
        pltpu.SemaphoreType.DMA,
    ],
)(_sc_router)


@jax.jit
def kernel(x, scale, per_expert_scale, W_proj):
    scale2d = scale.reshape(1, HIDDEN)
    probs = pl.pallas_call(
        _dense_block,
        grid=(TOKENS // BLOCK_T,),
        in_specs=[
            pl.BlockSpec((BLOCK_T, HIDDEN), lambda i: (i, 0)),
            pl.BlockSpec((1, HIDDEN), lambda i: (0, 0)),
            pl.BlockSpec((NUM_EXPERTS, HIDDEN), lambda i: (0, 0)),
        ],
        out_specs=pl.BlockSpec((BLOCK_T, NUM_EXPERTS), lambda i: (i, 0)),
        out_shape=jax.ShapeDtypeStruct((TOKENS, NUM_EXPERTS), jnp.float32),
    )(x, scale2d, W_proj)
    idx_flat, wgt_flat = _sc_router_full(probs, per_expert_scale)
    idx = idx_flat.reshape(TOKENS, TOP_K).astype(jnp.int64)
    wgt = wgt_flat.reshape(TOKENS, TOP_K)
    return idx, wgt


# speedup vs baseline: 1.2096x; 1.2096x over previous
import functools

import jax
import jax.numpy as jnp
from jax import lax
from jax.experimental import pallas as pl
from jax.experimental.pallas import tpu as pltpu
from jax.experimental.pallas import tpu_sc as plsc

HIDDEN = 2048
NUM_EXPERTS = 16
TOP_K = 2
EPS = 1e-6
TOKENS = 8192

BLOCK_T = 1024

_SC_INFO = plsc.get_sparse_core_info()
_NC, _NS, _NL = _SC_INFO.num_cores, _SC_INFO.num_subcores, _SC_INFO.num_lanes
_NW = _NC * _NS
_TOK_W = TOKENS // _NW
_GROUPS = _TOK_W // _NL


def _dense_body(xb, scale_row, w):
    ms = jnp.mean(xb * xb, axis=-1, keepdims=True)
    y = xb * lax.rsqrt(ms + EPS)
    y = y * scale_row
    y = y * (HIDDEN ** -0.5)
    logits = lax.dot_general(
        y, w,
        dimension_numbers=(((1,), (1,)), ((), ())),
        preferred_element_type=jnp.float32,
    )
    m = jnp.max(logits, axis=-1, keepdims=True)
    e = jnp.exp(logits - m)
    return e / jnp.sum(e, axis=-1, keepdims=True)


def _dense_block(x_ref, scale_ref, w_ref, p_ref):
    p_ref[...] = _dense_body(x_ref[...], scale_ref[...], w_ref[...])


_TOKW = TOKENS // _NW
_NGROUPS = _TOKW // _NL


def _sc_router(probs_hbm, pes_hbm, out_hbm,
               pv, pesv, outv, sem_p, sem_s):
    wid = lax.axis_index("s") * _NC + lax.axis_index("c")
    base = wid * _TOKW
    cp_p = pltpu.async_copy(probs_hbm.at[pl.ds(base, _TOKW), :], pv, sem_p)
    cp_s = pltpu.async_copy(pes_hbm, pesv, sem_s)
    cp_p.wait()
    cp_s.wait()
    lanes = lax.iota(jnp.int32, _NL)

    @plsc.parallel_loop(0, _NGROUPS, unroll=2)
    def group_body(g):
        rows = g * _NL + lanes
        m1 = jnp.full((_NL,), -jnp.inf, jnp.float32)
        m2 = jnp.full((_NL,), -jnp.inf, jnp.float32)
        i1 = jnp.zeros((_NL,), jnp.int32)
        i2 = jnp.zeros((_NL,), jnp.int32)
        for e in range(NUM_EXPERTS):
            v = plsc.load_gather(pv, [rows, jnp.full((_NL,), e, jnp.int32)])
            is1 = v > m1
            is2 = jnp.logical_not(is1) & (v > m2)
            m2 = jnp.where(is1, m1, jnp.where(is2, v, m2))
            i2 = jnp.where(is1, i1, jnp.where(is2, e, i2))
            m1 = jnp.where(is1, v, m1)
            i1 = jnp.where(is1, e, i1)
        g1 = plsc.load_gather(pesv, [i1])
        g2 = plsc.load_gather(pesv, [i2])
        s = m1 + m2
        sl = pl.ds(g * _NL, _NL)
        outv[0, sl] = plsc.bitcast(i1, jnp.float32)
        outv[1, sl] = plsc.bitcast(i2, jnp.float32)
        outv[2, sl] = (m1 / s) * g1
        outv[3, sl] = (m2 / s) * g2

    pltpu.sync_copy(outv, out_hbm.at[wid])


_sc_router_full = functools.partial(
    pl.kernel,
    out_type=jax.ShapeDtypeStruct((_NW, 4, _TOKW), jnp.float32),
    mesh=plsc.VectorSubcoreMesh(core_axis_name="c", subcore_axis_name="s"),
    compiler_params=pltpu.CompilerParams(needs_layout_passes=False),
    scratch_types=[
        pltpu.VMEM((_TOKW, NUM_EXPERTS), jnp.float32),
        pltpu.VMEM((NUM_EXPERTS,), jnp.float32),
        pltpu.VMEM((4, _TOKW), jnp.float32),
        pltpu.SemaphoreType.DMA,
        pltpu.SemaphoreType.DMA,
    ],
)(_sc_router)


@jax.jit
def kernel(x, scale, per_expert_scale, W_proj):
    scale2d = scale.reshape(1, HIDDEN)
    probs = pl.pallas_call(
        _dense_block,
        grid=(TOKENS // BLOCK_T,),
        in_specs=[
            pl.BlockSpec((BLOCK_T, HIDDEN), lambda i: (i, 0)),
            pl.BlockSpec((1, HIDDEN), lambda i: (0, 0)),
            pl.BlockSpec((NUM_EXPERTS, HIDDEN), lambda i: (0, 0)),
        ],
        out_specs=pl.BlockSpec((BLOCK_T, NUM_EXPERTS), lambda i: (i, 0)),
        out_shape=jax.ShapeDtypeStruct((TOKENS, NUM_EXPERTS), jnp.float32),
    )(x, scale2d, W_proj)
    packed = _sc_router_full(probs, per_expert_scale)
    i1 = packed[:, 0, :].reshape(-1).view(jnp.int32)
    i2 = packed[:, 1, :].reshape(-1).view(jnp.int32)
    w1 = packed[:, 2, :].reshape(-1)
    w2 = packed[:, 3, :].reshape(-1)
    idx = jnp.stack([i1, i2], axis=1).astype(jnp.int64)
    wgt = jnp.stack([w1, w2], axis=1)
    return idx, wgt
